# Initial kernel scaffold; baseline (speedup 1.0000x reference)
import jax
import jax.numpy as jnp
from jax.experimental import pallas as pl


def kernel(x_user, x_article, edge_index_reads, edge_index_rev, W_in_user, b_in_user, W_in_article, b_in_article, Wl1_reads, bl1_reads, Wr1_reads, Wl1_rev, bl1_rev, Wr1_rev, Wl2_reads, bl2_reads, Wr2_reads, Wl2_rev, bl2_rev, Wr2_rev, W_out, b_out):
    raise NotImplementedError("write your pallas kernel here")


# speedup vs baseline: 6.6283x; 6.6283x over previous
import functools

import jax
import jax.numpy as jnp
from jax import lax
from jax.experimental import pallas as pl
from jax.experimental.pallas import tpu as pltpu
from jax.experimental.pallas import tpu_sc as plsc

N_NODE = 50000
E = 800000
D_IN = 128
H = 64
HALF = H // 2
NC = 2
NS = 16
NW = NC * NS
IB = 80
KB = 5
EPS = E // NS
NOUTER = EPS // (IB * KB)
EROWS = E // IB
ROWS_PT = N_NODE // NS
FCH = 125

_sc_params = pltpu.CompilerParams(use_tc_tiling_on_sc=False,
                                  needs_layout_passes=False)


def _segsum_body(h_lo, h_hi, src2d, dst2d, zrows,
                 sum_out, acc, src_v, dst_v, rows_v, fbuf, sem):
    c = lax.axis_index("c")
    s = lax.axis_index("s")

    pltpu.sync_copy(zrows, fbuf)
    for j in range(ROWS_PT // FCH):
        pltpu.sync_copy(fbuf, acc.at[pl.ds(s * ROWS_PT + j * FCH, FCH)])
    plsc.subcore_barrier()

    def _run(h_half):
        def outer(i, carry):
            row0 = s * (EPS // IB) + i * KB
            pltpu.sync_copy(src2d.at[pl.ds(row0, KB)], src_v)
            pltpu.sync_copy(dst2d.at[pl.ds(row0, KB)], dst_v)
            descs = [
                pltpu.async_copy(h_half.at[src_v.at[k]], rows_v.at[k], sem)
                for k in range(KB)
            ]
            for d in descs:
                d.wait()
            for k in range(KB):
                pltpu.sync_copy(rows_v.at[k], acc.at[dst_v.at[k]], add=True)
            return carry
        lax.fori_loop(0, NOUTER, outer, 0)

    @pl.when(c == 0)
    def _lo():
        _run(h_lo)

    @pl.when(c == 1)
    def _hi():
        _run(h_hi)

    plsc.subcore_barrier()

    for j in range(ROWS_PT // FCH):
        r0 = s * ROWS_PT + j * FCH
        pltpu.sync_copy(acc.at[pl.ds(r0, FCH)], fbuf)
        pltpu.sync_copy(fbuf, sum_out.at[pl.ds(c * N_NODE + r0, FCH)])


_segsum = pl.kernel(
    _segsum_body,
    out_type=jax.ShapeDtypeStruct((NC * N_NODE, HALF), jnp.float32),
    mesh=plsc.VectorSubcoreMesh(core_axis_name="c", subcore_axis_name="s"),
    scratch_types=[
        pltpu.VMEM_SHARED((N_NODE, HALF), jnp.float32),
        pltpu.VMEM((KB, IB), jnp.int32),
        pltpu.VMEM((KB, IB), jnp.int32),
        pltpu.VMEM((KB, IB, HALF), jnp.float32),
        pltpu.VMEM((FCH, HALF), jnp.float32),
        pltpu.SemaphoreType.DMA,
    ],
    compiler_params=_sc_params,
)


CROWS = EROWS // NW
CKB = 4


def _cnt_body(dstr2d, dstv2d, zcnt, cntr_out, cntv_out, cnt_v, idx_v):
    c = lax.axis_index("c")
    s = lax.axis_index("s")
    w = s * NC + c
    ones16 = jnp.full((16,), 1.0, jnp.float32)

    def _hist(dst2d, out):
        pltpu.sync_copy(zcnt, cnt_v)

        def body(i, carry):
            pltpu.sync_copy(dst2d.at[pl.ds(w * CROWS + i * CKB, CKB)], idx_v)
            for k in range(CKB):
                for t in range(IB // 16):
                    idx16 = idx_v[k, pl.ds(t * 16, 16)]
                    plsc.addupdate_scatter(cnt_v, [idx16], ones16)
            return carry
        lax.fori_loop(0, CROWS // CKB, body, 0)

        @pl.when(w < EROWS - NW * CROWS)
        def _tail():
            pltpu.sync_copy(dst2d.at[pl.ds(NW * CROWS + w, 1)],
                            idx_v.at[pl.ds(0, 1)])
            for t in range(IB // 16):
                idx16 = idx_v[0, pl.ds(t * 16, 16)]
                plsc.addupdate_scatter(cnt_v, [idx16], ones16)

        pltpu.sync_copy(cnt_v, out.at[w])

    _hist(dstr2d, cntr_out)
    _hist(dstv2d, cntv_out)


_cnt_kernel = pl.kernel(
    _cnt_body,
    out_type=(jax.ShapeDtypeStruct((NW, N_NODE), jnp.float32),
              jax.ShapeDtypeStruct((NW, N_NODE), jnp.float32)),
    mesh=plsc.VectorSubcoreMesh(core_axis_name="c", subcore_axis_name="s"),
    scratch_types=[
        pltpu.VMEM((N_NODE,), jnp.float32),
        pltpu.VMEM((CKB, IB), jnp.int32),
    ],
    compiler_params=_sc_params,
)


def _segsum_call(h, src2d, dst2d):
    zrows = jnp.zeros((FCH, HALF), jnp.float32)
    return _segsum(h[:, :HALF], h[:, HALF:], src2d, dst2d, zrows)


_RB = 2000


def _proj_relu(x, W, b):
    n, d = x.shape
    h = W.shape[0]

    def body(x_ref, w_ref, b_ref, o_ref):
        y = jnp.dot(x_ref[...], w_ref[...].T,
                    preferred_element_type=jnp.float32) + b_ref[...]
        o_ref[...] = jnp.maximum(y, 0.0)

    return pl.pallas_call(
        body,
        grid=(n // _RB,),
        in_specs=[
            pl.BlockSpec((_RB, d), lambda i: (i, 0)),
            pl.BlockSpec((h, d), lambda i: (0, 0)),
            pl.BlockSpec((1, h), lambda i: (0, 0)),
        ],
        out_specs=pl.BlockSpec((_RB, h), lambda i: (i, 0)),
        out_shape=jax.ShapeDtypeStruct((n, h), jnp.float32),
    )(x, W, b.reshape(1, h))


def _combine(sums, cntT, xdst, Wl, bl, Wr, relu, Wout=None, bout=None):
    n = xdst.shape[0]
    nb = n // _RB
    out_h = 2 if Wout is not None else H

    def body(slo_ref, shi_ref, cp_ref, xd_ref, wl_ref, bl_ref, wr_ref,
             *rest):
        if Wout is not None:
            wo_ref, bo_ref, o_ref = rest
        else:
            (o_ref,) = rest
        cnt = jnp.sum(cp_ref[...], axis=1)
        inv = 1.0 / jnp.maximum(cnt, 1.0)
        sm = jnp.concatenate([slo_ref[...], shi_ref[...]], axis=1)
        sm = sm * inv[:, None]
        y = (jnp.dot(sm, wl_ref[...].T, preferred_element_type=jnp.float32)
             + bl_ref[...]
             + jnp.dot(xd_ref[...], wr_ref[...].T,
                       preferred_element_type=jnp.float32))
        if relu:
            y = jnp.maximum(y, 0.0)
        if Wout is not None:
            y = jnp.dot(y, wo_ref[...].T,
                        preferred_element_type=jnp.float32) + bo_ref[...]
        o_ref[...] = y

    in_specs = [
        pl.BlockSpec((_RB, HALF), lambda i: (i, 0)),
        pl.BlockSpec((_RB, HALF), lambda i: (i + nb, 0)),
        pl.BlockSpec((_RB, NW), lambda i: (i, 0)),
        pl.BlockSpec((_RB, H), lambda i: (i, 0)),
        pl.BlockSpec((H, H), lambda i: (0, 0)),
        pl.BlockSpec((1, H), lambda i: (0, 0)),
        pl.BlockSpec((H, H), lambda i: (0, 0)),
    ]
    args = [sums, sums, cntT, xdst, Wl, bl.reshape(1, H), Wr]
    if Wout is not None:
        in_specs += [
            pl.BlockSpec((2, H), lambda i: (0, 0)),
            pl.BlockSpec((1, 2), lambda i: (0, 0)),
        ]
        args += [Wout, bout.reshape(1, 2)]

    return pl.pallas_call(
        body,
        grid=(nb,),
        in_specs=in_specs,
        out_specs=pl.BlockSpec((_RB, out_h), lambda i: (i, 0)),
        out_shape=jax.ShapeDtypeStruct((n, out_h), jnp.float32),
    )(*args)


def kernel(x_user, x_article, edge_index_reads, edge_index_rev, W_in_user,
           b_in_user, W_in_article, b_in_article, Wl1_reads, bl1_reads,
           Wr1_reads, Wl1_rev, bl1_rev, Wr1_rev, Wl2_reads, bl2_reads,
           Wr2_reads, Wl2_rev, bl2_rev, Wr2_rev, W_out, b_out):
    srcr2d = edge_index_reads[0].reshape(EROWS, IB)
    dstr2d = edge_index_reads[1].reshape(EROWS, IB)
    srcv2d = edge_index_rev[0].reshape(EROWS, IB)
    dstv2d = edge_index_rev[1].reshape(EROWS, IB)

    zcnt = jnp.zeros((N_NODE,), jnp.float32)
    cntr, cntv = _cnt_kernel(dstr2d, dstv2d, zcnt)
    cntrT, cntvT = cntr.T, cntv.T

    h_u = _proj_relu(x_user, W_in_user, b_in_user)
    h_a = _proj_relu(x_article, W_in_article, b_in_article)

    sum1a = _segsum_call(h_u, srcr2d, dstr2d)
    sum1u = _segsum_call(h_a, srcv2d, dstv2d)
    h1_a = _combine(sum1a, cntrT, h_a, Wl1_reads, bl1_reads, Wr1_reads, True)
    h1_u = _combine(sum1u, cntvT, h_u, Wl1_rev, bl1_rev, Wr1_rev, True)

    sum2a = _segsum_call(h1_u, srcr2d, dstr2d)
    out = _combine(sum2a, cntrT, h1_a, Wl2_reads, bl2_reads, Wr2_reads,
                   False, Wout=W_out, bout=b_out)
    return out
